# baseline (speedup 1.0000x reference)
import functools

import jax
import jax.numpy as jnp
from jax import lax
from jax.experimental import pallas as pl
from jax.experimental.pallas import tpu as pltpu
from jax.experimental.pallas import tpu_sc as plsc

VOCAB = 1000
HIDDEN = 32
HPAD = 128
B, L = 4096, 20
NTOK = B * L

NC, NS = 2, 16
NW = NC * NS
K = 128
NBUF = 2

L_H = L // 2
NTOK_H = B * L_H


def _make_gather(ntok):
    bpw = ntok // NW
    nch = bpw // K

    def body(emb_hbm, idx_hbm, h_hbm, idx_v, bufs, sems):
        wid = lax.axis_index("s") * NC + lax.axis_index("c")
        pltpu.sync_copy(idx_hbm.at[pl.ds(wid * nch, nch)], idx_v)
        base = wid * bpw

        pltpu.async_copy(emb_hbm.at[idx_v.at[0]], bufs.at[0], sems.at[0])

        def loop(g, carry):
            for s in range(NBUF):
                c = g * NBUF + s
                pltpu.make_async_copy(
                    emb_hbm.at[idx_v.at[c]], bufs.at[s], sems.at[s]).wait()
                nxt = (s + 1) % NBUF

                @pl.when(c + 1 < nch)
                def _():
                    pltpu.async_copy(
                        emb_hbm.at[idx_v.at[c + 1]], bufs.at[nxt], sems.at[nxt])

                pltpu.sync_copy(bufs.at[s], h_hbm.at[pl.ds(base + c * K, K)])
            return carry

        lax.fori_loop(0, nch // NBUF, loop, 0)

    return pl.kernel(
        body,
        out_type=jax.ShapeDtypeStruct((ntok, HPAD), jnp.float32),
        mesh=plsc.VectorSubcoreMesh(core_axis_name="c", subcore_axis_name="s"),
        scratch_types=[
            pltpu.VMEM((nch, K), jnp.int32),
            pltpu.VMEM((NBUF, K, HPAD), jnp.float32),
            pltpu.SemaphoreType.DMA((NBUF,)),
        ],
        compiler_params=pltpu.CompilerParams(use_tc_tiling_on_sc=False),
    )


_sc_gather_half = _make_gather(NTOK_H)


def _mm_body_a(w_ref, h_ref, b_ref, out_ref):
    acc = lax.dot_general(
        w_ref[...], h_ref[...],
        dimension_numbers=(((1,), (1,)), ((), ())),
        preferred_element_type=jnp.float32,
    ) + b_ref[...]
    out_ref[...] = acc[None]


def _mm_body_b(w_ref, h_ref, b_ref, prev_ref, out_ref):
    del prev_ref
    _mm_body_a(w_ref, h_ref, b_ref, out_ref)


_mm_a = pl.pallas_call(
    _mm_body_a,
    grid=(L_H,),
    in_specs=[
        pl.BlockSpec((VOCAB, HPAD), lambda l: (0, 0)),
        pl.BlockSpec((B, HPAD), lambda l: (l, 0)),
        pl.BlockSpec((VOCAB, 1), lambda l: (0, 0)),
    ],
    out_specs=pl.BlockSpec((1, VOCAB, B), lambda l: (l, 0, 0)),
    out_shape=jax.ShapeDtypeStruct((L, VOCAB, B), jnp.float32),
    compiler_params=pltpu.CompilerParams(
        dimension_semantics=("arbitrary",),
        vmem_limit_bytes=60 * 1024 * 1024,
    ),
)

_mm_b = pl.pallas_call(
    _mm_body_b,
    grid=(L_H,),
    in_specs=[
        pl.BlockSpec((VOCAB, HPAD), lambda l: (0, 0)),
        pl.BlockSpec((B, HPAD), lambda l: (l, 0)),
        pl.BlockSpec((VOCAB, 1), lambda l: (0, 0)),
        pl.BlockSpec(memory_space=pltpu.MemorySpace.HBM),
    ],
    out_specs=pl.BlockSpec((1, VOCAB, B), lambda l: (l + L_H, 0, 0)),
    out_shape=jax.ShapeDtypeStruct((L, VOCAB, B), jnp.float32),
    input_output_aliases={3: 0},
    compiler_params=pltpu.CompilerParams(
        dimension_semantics=("arbitrary",),
        vmem_limit_bytes=60 * 1024 * 1024,
    ),
)


@jax.jit
def kernel(x, emb_table, W, b):
    emb_pad = jnp.zeros((VOCAB, HPAD), jnp.float32).at[:, :HIDDEN].set(emb_table)
    w_pad = jnp.zeros((VOCAB, HPAD), jnp.float32).at[:, :HIDDEN].set(W)
    b2 = b.reshape(VOCAB, 1)
    idx_t = x.T
    idx_a = idx_t[:L_H].reshape(NTOK_H // K, K)
    idx_b = idx_t[L_H:].reshape(NTOK_H // K, K)
    h_a = _sc_gather_half(emb_pad, idx_a)
    h_b = _sc_gather_half(emb_pad, idx_b)
    out1 = _mm_a(w_pad, h_a, b2)
    out_phys = _mm_b(w_pad, h_b, b2, out1)
    return out_phys.transpose(2, 0, 1)


# speedup vs baseline: 1.0205x; 1.0205x over previous
import functools

import jax
import jax.numpy as jnp
from jax import lax
from jax.experimental import pallas as pl
from jax.experimental.pallas import tpu as pltpu
from jax.experimental.pallas import tpu_sc as plsc

VOCAB = 1000
HIDDEN = 32
HPAD = 128
B, L = 4096, 20
NTOK = B * L

NC, NS = 2, 16
NW = NC * NS
BPW = NTOK // NW
K = 128
NCH = BPW // K
NBUF = 4


def _sc_gather_body(emb_hbm, idx_hbm, h_hbm, idx_v, bufs, sems):
    wid = lax.axis_index("s") * NC + lax.axis_index("c")
    pltpu.sync_copy(idx_hbm.at[pl.ds(wid * NCH, NCH)], idx_v)
    base = wid * BPW

    for s in range(NBUF):
        pltpu.async_copy(emb_hbm.at[idx_v.at[s]], bufs.at[s], sems.at[s])

    def body(g, carry):
        for s in range(NBUF):
            c = g * NBUF + s
            pltpu.make_async_copy(
                emb_hbm.at[idx_v.at[c]], bufs.at[s], sems.at[s]).wait()
            pltpu.sync_copy(bufs.at[s], h_hbm.at[pl.ds(base + c * K, K)])

            @pl.when(c + NBUF < NCH)
            def _():
                pltpu.async_copy(
                    emb_hbm.at[idx_v.at[c + NBUF]], bufs.at[s], sems.at[s])
        return carry

    lax.fori_loop(0, NCH // NBUF, body, 0)


_sc_gather = functools.partial(
    pl.kernel,
    out_type=jax.ShapeDtypeStruct((NTOK, HPAD), jnp.float32),
    mesh=plsc.VectorSubcoreMesh(core_axis_name="c", subcore_axis_name="s"),
    scratch_types=[
        pltpu.VMEM((NCH, K), jnp.int32),
        pltpu.VMEM((NBUF, K, HPAD), jnp.float32),
        pltpu.SemaphoreType.DMA((NBUF,)),
    ],
    compiler_params=pltpu.CompilerParams(use_tc_tiling_on_sc=False),
)(_sc_gather_body)


def _mm_body(w_ref, h_ref, b_ref, out_ref):
    acc = lax.dot_general(
        w_ref[...], h_ref[...],
        dimension_numbers=(((1,), (1,)), ((), ())),
        preferred_element_type=jnp.float32,
    ) + b_ref[...]
    out_ref[...] = acc[None]


_mm_call = pl.pallas_call(
    _mm_body,
    grid=(L,),
    in_specs=[
        pl.BlockSpec((VOCAB, HPAD), lambda l: (0, 0)),
        pl.BlockSpec((B, HPAD), lambda l: (l, 0)),
        pl.BlockSpec((VOCAB, 1), lambda l: (0, 0)),
    ],
    out_specs=pl.BlockSpec((1, VOCAB, B), lambda l: (l, 0, 0)),
    out_shape=jax.ShapeDtypeStruct((L, VOCAB, B), jnp.float32),
    compiler_params=pltpu.CompilerParams(
        dimension_semantics=("arbitrary",),
        vmem_limit_bytes=60 * 1024 * 1024,
    ),
)


@jax.jit
def kernel(x, emb_table, W, b):
    emb_pad = jnp.zeros((VOCAB, HPAD), jnp.float32).at[:, :HIDDEN].set(emb_table)
    w_pad = jnp.zeros((VOCAB, HPAD), jnp.float32).at[:, :HIDDEN].set(W)
    idx = x.T.reshape(NTOK // K, K)
    h = _sc_gather(emb_pad, idx)
    out_phys = _mm_call(w_pad, h, b.reshape(VOCAB, 1))
    return out_phys.transpose(2, 0, 1)
